# baseline (speedup 1.0000x reference)
import jax
import jax.numpy as jnp
from jax.experimental import pallas as pl
from jax.experimental.pallas import tpu as pltpu
from functools import partial

_B, _S, _H = 32, 8, 1024
_E = 64
_TOP_K = 8
_N_GROUP = 8
_TOPK_GROUP = 4
_I_MOE = 512
_I_SHARED = 2048
_SCALING = 2.5
_T = _B * _S
_GSZ = _E // _N_GROUP
_SH_FIRST = 1
_SH_STEPS = 8
_SH_CHUNK = _I_SHARED // _SH_STEPS

_NEG = -1e30


def _routing(x, rw, eb):
    logits = jax.lax.dot_general(
        x, rw, (((1,), (1,)), ((), ())), preferred_element_type=jnp.float32)
    scores = jax.nn.sigmoid(logits)
    sfc = scores + eb
    lane = jax.lax.broadcasted_iota(jnp.int32, (_T, _E), 1)

    gs = []
    for g in range(_N_GROUP):
        seg = sfc[:, g * _GSZ:(g + 1) * _GSZ]
        il = jax.lax.broadcasted_iota(jnp.int32, (_T, _GSZ), 1)
        m1 = jnp.max(seg, axis=1, keepdims=True)
        fi = jnp.min(jnp.where(seg == m1, il, 127), axis=1, keepdims=True)
        m2 = jnp.max(jnp.where(il == fi, _NEG, seg), axis=1, keepdims=True)
        gs.append(m1 + m2)
    group_scores = jnp.concatenate(gs, axis=1)

    gil = jax.lax.broadcasted_iota(jnp.int32, (_T, _N_GROUP), 1)
    gmask = jnp.zeros((_T, _N_GROUP), jnp.float32)
    gtmp = group_scores
    for _ in range(_TOPK_GROUP):
        m = jnp.max(gtmp, axis=1, keepdims=True)
        fi = jnp.min(jnp.where(gtmp == m, gil, 127), axis=1, keepdims=True)
        sel = gil == fi
        gmask = jnp.where(sel, 1.0, gmask)
        gtmp = jnp.where(sel, _NEG, gtmp)

    smask = jnp.concatenate(
        [jnp.broadcast_to(gmask[:, g:g + 1], (_T, _GSZ)) for g in range(_N_GROUP)],
        axis=1)
    masked = jnp.where(smask > 0, sfc, 0.0)

    comb = jnp.zeros((_T, _E), jnp.float32)
    wsum = jnp.zeros((_T, 1), jnp.float32)
    for _ in range(_TOP_K):
        m = jnp.max(masked, axis=1, keepdims=True)
        fi = jnp.min(jnp.where(masked == m, lane, 9999), axis=1, keepdims=True)
        sel = lane == fi
        w = jnp.sum(jnp.where(sel, scores, 0.0), axis=1, keepdims=True)
        comb = comb + jnp.where(sel, w, 0.0)
        wsum = wsum + w
        masked = jnp.where(sel, _NEG, masked)
    return comb * (_SCALING / (wsum + 1e-20))


_EPG = 4


_MTILE = 64


def _moe_body(x_ref, rw_ref, eb_ref, up_ref, dn_ref,
              su_ref, sd_ref, out_ref, comb_ref, pos_ref):
    e = pl.program_id(0)
    x = x_ref[...]

    @pl.when(e == 0)
    def _init():
        comb = _routing(x, rw_ref[...], eb_ref[...])
        comb_ref[...] = comb
        r_io = jax.lax.broadcasted_iota(jnp.int32, (_T, _T), 0)
        c_io = jax.lax.broadcasted_iota(jnp.int32, (_T, _T), 1)
        ltri = (r_io >= c_io).astype(jnp.float32)
        pos_ref[...] = jax.lax.dot_general(
            ltri, (comb > 0).astype(jnp.float32), (((1,), (0,)), ((), ())),
            preferred_element_type=jnp.float32)
        out_ref[...] = jnp.zeros_like(out_ref)

    xb = x.astype(jnp.bfloat16)

    @pl.when((e >= _SH_FIRST) & (e < _SH_FIRST + _SH_STEPS))
    def _shared():
        hs = jnp.maximum(jax.lax.dot_general(
            xb, su_ref[...].astype(jnp.bfloat16), (((1,), (1,)), ((), ())),
            preferred_element_type=jnp.float32), 0.0)
        out_ref[...] += jax.lax.dot_general(
            hs.astype(jnp.bfloat16), sd_ref[...].astype(jnp.bfloat16),
            (((1,), (1,)), ((), ())),
            preferred_element_type=jnp.float32)

    lane = jax.lax.broadcasted_iota(jnp.int32, (_T, _E), 1)
    slot = jax.lax.broadcasted_iota(jnp.int32, (_T, _MTILE), 1).astype(jnp.float32)
    for j in range(_EPG):
        ej = e * _EPG + j
        sel_e = lane == ej
        c = jnp.sum(jnp.where(sel_e, comb_ref[...], 0.0),
                    axis=1, keepdims=True)
        posc = jnp.sum(jnp.where(sel_e, pos_ref[...], 0.0),
                       axis=1, keepdims=True)
        cnt = jnp.max(posc)
        member = c > 0
        for m in range(_T // _MTILE):
            base = float(m * _MTILE)

            @pl.when(base < cnt)
            def _tile(j=j, m=m, base=base, c=c, posc=posc, member=member):
                g = jnp.where(member & (posc - 1.0 - base == slot),
                              1.0, 0.0)
                gb = g.astype(jnp.bfloat16)
                xg = jax.lax.dot_general(
                    gb, xb, (((0,), (0,)), ((), ())),
                    preferred_element_type=jnp.float32)
                ws = jax.lax.dot_general(
                    g, c, (((0,), (0,)), ((), ())),
                    preferred_element_type=jnp.float32)
                hg = jnp.maximum(jax.lax.dot_general(
                    xg.astype(jnp.bfloat16), up_ref[j].astype(jnp.bfloat16),
                    (((1,), (1,)), ((), ())),
                    preferred_element_type=jnp.float32), 0.0)
                og = jax.lax.dot_general(
                    (hg * ws).astype(jnp.bfloat16),
                    dn_ref[j].astype(jnp.bfloat16), (((1,), (1,)), ((), ())),
                    preferred_element_type=jnp.float32)
                out_ref[...] += jax.lax.dot_general(
                    g, og, (((1,), (0,)), ((), ())),
                    preferred_element_type=jnp.float32)


def kernel(hidden_states, router_weight, up_w, down_w,
           shared_up_w, shared_down_w, e_bias):
    x = hidden_states.reshape(_T, _H)
    eb = e_bias.reshape(1, _E)

    out = pl.pallas_call(
        _moe_body,
        grid=(_E // _EPG,),
        in_specs=[
            pl.BlockSpec((_T, _H), lambda e: (0, 0)),
            pl.BlockSpec((_E, _H), lambda e: (0, 0)),
            pl.BlockSpec((1, _E), lambda e: (0, 0)),
            pl.BlockSpec((_EPG, _I_MOE, _H), lambda e: (e, 0, 0)),
            pl.BlockSpec((_EPG, _H, _I_MOE), lambda e: (e, 0, 0)),
            pl.BlockSpec((_SH_CHUNK, _H),
                         lambda e: (jnp.clip(e - _SH_FIRST, 0,
                                             _SH_STEPS - 1), 0)),
            pl.BlockSpec((_H, _SH_CHUNK),
                         lambda e: (0, jnp.clip(e - _SH_FIRST, 0,
                                                _SH_STEPS - 1))),
        ],
        out_specs=pl.BlockSpec((_T, _H), lambda e: (0, 0)),
        out_shape=jax.ShapeDtypeStruct((_T, _H), jnp.float32),
        scratch_shapes=[pltpu.VMEM((_T, _E), jnp.float32),
                        pltpu.VMEM((_T, _E), jnp.float32)],
    )(x, router_weight, eb, up_w, down_w, shared_up_w, shared_down_w)

    return out.reshape(_B, _S, _H)


# speedup vs baseline: 1.1111x; 1.1111x over previous
import jax
import jax.numpy as jnp
from jax.experimental import pallas as pl
from jax.experimental.pallas import tpu as pltpu

_B, _S, _H = 32, 8, 1024
_E = 64
_TOP_K = 8
_N_GROUP = 8
_TOPK_GROUP = 4
_I_MOE = 512
_I_SHARED = 2048
_SCALING = 2.5
_T = _B * _S
_GSZ = _E // _N_GROUP
_SH_FIRST = 1
_SH_STEPS = 8
_SH_CHUNK = _I_SHARED // _SH_STEPS

_NEG = -1e30


def _routing(x, rw, eb):
    logits = jax.lax.dot_general(
        x, rw, (((1,), (1,)), ((), ())), preferred_element_type=jnp.float32)
    scores = jax.nn.sigmoid(logits)
    sfc = scores + eb
    lane = jax.lax.broadcasted_iota(jnp.int32, (_T, _E), 1)

    gs = []
    for g in range(_N_GROUP):
        seg = sfc[:, g * _GSZ:(g + 1) * _GSZ]
        il = jax.lax.broadcasted_iota(jnp.int32, (_T, _GSZ), 1)
        m1 = jnp.max(seg, axis=1, keepdims=True)
        fi = jnp.min(jnp.where(seg == m1, il, 127), axis=1, keepdims=True)
        m2 = jnp.max(jnp.where(il == fi, _NEG, seg), axis=1, keepdims=True)
        gs.append(m1 + m2)
    group_scores = jnp.concatenate(gs, axis=1)

    gil = jax.lax.broadcasted_iota(jnp.int32, (_T, _N_GROUP), 1)
    gmask = jnp.zeros((_T, _N_GROUP), jnp.float32)
    gtmp = group_scores
    for _ in range(_TOPK_GROUP):
        m = jnp.max(gtmp, axis=1, keepdims=True)
        fi = jnp.min(jnp.where(gtmp == m, gil, 127), axis=1, keepdims=True)
        sel = gil == fi
        gmask = jnp.where(sel, 1.0, gmask)
        gtmp = jnp.where(sel, _NEG, gtmp)

    smask = jnp.concatenate(
        [jnp.broadcast_to(gmask[:, g:g + 1], (_T, _GSZ)) for g in range(_N_GROUP)],
        axis=1)
    masked = jnp.where(smask > 0, sfc, 0.0)

    comb = jnp.zeros((_T, _E), jnp.float32)
    wsum = jnp.zeros((_T, 1), jnp.float32)
    for _ in range(_TOP_K):
        m = jnp.max(masked, axis=1, keepdims=True)
        fi = jnp.min(jnp.where(masked == m, lane, 9999), axis=1, keepdims=True)
        sel = lane == fi
        w = jnp.sum(jnp.where(sel, scores, 0.0), axis=1, keepdims=True)
        comb = comb + jnp.where(sel, w, 0.0)
        wsum = wsum + w
        masked = jnp.where(sel, _NEG, masked)
    return comb * (_SCALING / (wsum + 1e-20))


_EPG = 4


def _moe_body(x_ref, rw_ref, eb_ref, up_ref, dn_ref,
              su_ref, sd_ref, out_ref, comb_ref):
    e = pl.program_id(0)
    x = x_ref[...]

    @pl.when(e == 0)
    def _init():
        comb_ref[...] = _routing(x, rw_ref[...], eb_ref[...])
        out_ref[...] = jnp.zeros_like(out_ref)

    xb = x.astype(jnp.bfloat16)

    @pl.when((e >= _SH_FIRST) & (e < _SH_FIRST + _SH_STEPS))
    def _shared():
        hs = jnp.maximum(jax.lax.dot_general(
            xb, su_ref[...].astype(jnp.bfloat16), (((1,), (1,)), ((), ())),
            preferred_element_type=jnp.float32), 0.0)
        out_ref[...] += jax.lax.dot_general(
            hs.astype(jnp.bfloat16), sd_ref[...].astype(jnp.bfloat16),
            (((1,), (1,)), ((), ())),
            preferred_element_type=jnp.float32)

    lane = jax.lax.broadcasted_iota(jnp.int32, (_T, _E), 1)
    acc = out_ref[...]
    for j in range(_EPG):
        ej = e * _EPG + j
        c = jnp.sum(jnp.where(lane == ej, comb_ref[...], 0.0),
                    axis=1, keepdims=True)
        h = jnp.maximum(jax.lax.dot_general(
            xb, up_ref[j].astype(jnp.bfloat16), (((1,), (1,)), ((), ())),
            preferred_element_type=jnp.float32), 0.0)
        acc += jax.lax.dot_general(
            (h * c).astype(jnp.bfloat16), dn_ref[j].astype(jnp.bfloat16),
            (((1,), (1,)), ((), ())),
            preferred_element_type=jnp.float32)
    out_ref[...] = acc


def kernel(hidden_states, router_weight, up_w, down_w,
           shared_up_w, shared_down_w, e_bias):
    x = hidden_states.reshape(_T, _H)
    eb = e_bias.reshape(1, _E)

    out = pl.pallas_call(
        _moe_body,
        grid=(_E // _EPG,),
        in_specs=[
            pl.BlockSpec((_T, _H), lambda e: (0, 0)),
            pl.BlockSpec((_E, _H), lambda e: (0, 0)),
            pl.BlockSpec((1, _E), lambda e: (0, 0)),
            pl.BlockSpec((_EPG, _I_MOE, _H), lambda e: (e, 0, 0)),
            pl.BlockSpec((_EPG, _H, _I_MOE), lambda e: (e, 0, 0)),
            pl.BlockSpec((_SH_CHUNK, _H),
                         lambda e: (jnp.clip(e - _SH_FIRST, 0,
                                             _SH_STEPS - 1), 0)),
            pl.BlockSpec((_H, _SH_CHUNK),
                         lambda e: (0, jnp.clip(e - _SH_FIRST, 0,
                                                _SH_STEPS - 1))),
        ],
        out_specs=pl.BlockSpec((_T, _H), lambda e: (0, 0)),
        out_shape=jax.ShapeDtypeStruct((_T, _H), jnp.float32),
        scratch_shapes=[pltpu.VMEM((_T, _E), jnp.float32)],
    )(x, router_weight, eb, up_w, down_w, shared_up_w, shared_down_w)

    return out.reshape(_B, _S, _H)
